# baseline (speedup 1.0000x reference)
import functools

import jax
import jax.numpy as jnp
from jax import lax
from jax.experimental import pallas as pl
from jax.experimental.pallas import tpu as pltpu
from jax.experimental.pallas import tpu_sc as plsc

B = 16384
D = 64
PD = 2 * D
V = 1000000
NC = 2
NS = 16
NW = NC * NS
BPW = B // NW
HALF = BPW // 2


def _sc_pair_gather(user_ids, item_ids, ut2, it2):
    @functools.partial(
        pl.kernel,
        mesh=plsc.VectorSubcoreMesh(core_axis_name="c", subcore_axis_name="s"),
        out_type=[
            jax.ShapeDtypeStruct((B, PD), jnp.float32),
            jax.ShapeDtypeStruct((B, PD), jnp.float32),
        ],
        scratch_types=[
            pltpu.VMEM((BPW,), jnp.int32),
            pltpu.VMEM((BPW,), jnp.int32),
            pltpu.VMEM((HALF, PD), jnp.float32),
            pltpu.VMEM((HALF, PD), jnp.float32),
            pltpu.SemaphoreType.DMA,
        ],
    )
    def k(uids_hbm, iids_hbm, utab_hbm, itab_hbm, uout_hbm, iout_hbm,
          uidx_v, iidx_v, urows_v, irows_v, sem):
        wid = lax.axis_index("s") * NC + lax.axis_index("c")
        base = wid * BPW
        pltpu.sync_copy(uids_hbm.at[pl.ds(base, BPW)], uidx_v)
        pltpu.sync_copy(iids_hbm.at[pl.ds(base, BPW)], iidx_v)

        for h in range(2):
            hoff = h * HALF

            def group(g, carry):
                goff = g * 16
                uv = uidx_v[pl.ds(hoff + goff, 16)]
                iv = iidx_v[pl.ds(hoff + goff, 16)]
                for l in range(16):
                    pltpu.async_copy(
                        utab_hbm.at[pl.ds(lax.shift_right_logical(uv[l], 1),
                                          1), :],
                        urows_v.at[pl.ds(goff + l, 1), :], sem)
                    pltpu.async_copy(
                        itab_hbm.at[pl.ds(lax.shift_right_logical(iv[l], 1),
                                          1), :],
                        irows_v.at[pl.ds(goff + l, 1), :], sem)
                return carry

            lax.fori_loop(0, HALF // 16, group, 0)
            pltpu.make_async_copy(utab_hbm.at[pl.ds(0, HALF), :], urows_v,
                                  sem).wait()
            pltpu.make_async_copy(itab_hbm.at[pl.ds(0, HALF), :], irows_v,
                                  sem).wait()
            pltpu.sync_copy(urows_v,
                            uout_hbm.at[pl.ds(base + hoff, HALF), :])
            pltpu.sync_copy(irows_v,
                            iout_hbm.at[pl.ds(base + hoff, HALF), :])

    return k(user_ids, item_ids, ut2, it2)


def _tc_towers(u_pair, i_pair, u_par, i_par, W_u, b_u, W_i, b_i):
    BLK = 2048

    def body(u_ref, i_ref, up_ref, ip_ref, wu_ref, bu_ref, wi_ref, bi_ref,
             out_ref):
        up = up_ref[...][:, None]
        ip = ip_ref[...][:, None]
        u_emb = jnp.where(up == 0, u_ref[:, :D], u_ref[:, D:])
        i_emb = jnp.where(ip == 0, i_ref[:, :D], i_ref[:, D:])
        u = jnp.dot(u_emb, wu_ref[...],
                    preferred_element_type=jnp.float32) + bu_ref[...]
        i = jnp.dot(i_emb, wi_ref[...],
                    preferred_element_type=jnp.float32) + bi_ref[...]
        u = jnp.maximum(u, 0.0)
        i = jnp.maximum(i, 0.0)
        out_ref[...] = jnp.sum(u * i, axis=1)

    return pl.pallas_call(
        body,
        grid=(B // BLK,),
        in_specs=[
            pl.BlockSpec((BLK, PD), lambda g: (g, 0)),
            pl.BlockSpec((BLK, PD), lambda g: (g, 0)),
            pl.BlockSpec((BLK,), lambda g: (g,)),
            pl.BlockSpec((BLK,), lambda g: (g,)),
            pl.BlockSpec((D, D), lambda g: (0, 0)),
            pl.BlockSpec((D,), lambda g: (0,)),
            pl.BlockSpec((D, D), lambda g: (0, 0)),
            pl.BlockSpec((D,), lambda g: (0,)),
        ],
        out_specs=pl.BlockSpec((BLK,), lambda g: (g,)),
        out_shape=jax.ShapeDtypeStruct((B,), jnp.float32),
    )(u_pair, i_pair, u_par, i_par, W_u, b_u, W_i, b_i)


def kernel(user_ids, item_ids, user_table, item_table, W_u, b_u, W_i, b_i):
    ut2 = user_table.reshape(V // 2, PD)
    it2 = item_table.reshape(V // 2, PD)
    u_par = jnp.bitwise_and(user_ids, 1)
    i_par = jnp.bitwise_and(item_ids, 1)
    u_pair, i_pair = _sc_pair_gather(user_ids, item_ids, ut2, it2)
    return _tc_towers(u_pair, i_pair, u_par, i_par, W_u, b_u, W_i, b_i)


# speedup vs baseline: 1.4087x; 1.4087x over previous
import functools

import jax
import jax.numpy as jnp
from jax import lax
from jax.experimental import pallas as pl
from jax.experimental.pallas import tpu as pltpu
from jax.experimental.pallas import tpu_sc as plsc

B = 16384
D = 64
V = 1000000
NC = 2
NS = 16
NW = NC * NS
BLKW = 128
NBLK = V // BLKW
TAILW = V - NBLK * BLKW
NPAIR = NBLK // 2
NEDGE = NBLK + 2
NBPAD = 7840
RING = 128


def _sc_scan_gather(su, pu, bu, si, pi, bi, ut, it, utail, itail):
    @functools.partial(
        pl.kernel,
        mesh=plsc.VectorSubcoreMesh(core_axis_name="c", subcore_axis_name="s"),
        out_type=[
            jax.ShapeDtypeStruct((B, D), jnp.float32),
            jax.ShapeDtypeStruct((B, D), jnp.float32),
        ],
        scratch_types=[
            pltpu.VMEM((B + 16,), jnp.int32),
            pltpu.VMEM((B + 16,), jnp.int32),
            pltpu.VMEM((NBPAD + 16,), jnp.int32),
            pltpu.VMEM((D + 1, BLKW), jnp.float32),
            pltpu.VMEM((D + 1, BLKW), jnp.float32),
            pltpu.VMEM((TAILW, D), jnp.float32),
            pltpu.VMEM((RING, D), jnp.float32),
            pltpu.SemaphoreType.DMA,
            pltpu.SemaphoreType.DMA,
            pltpu.SemaphoreType.DMA,
        ],
    )
    def k(su_h, pu_h, bu_h, si_h, pi_h, bi_h, ut_h, it_h, utail_h, itail_h,
          uout_h, iout_h,
          ids_v, pos_v, bnd_v, bufa, bufb, buft, ring, sema, semb, semw):
        wid = lax.axis_index("s") * NC + lax.axis_index("c")
        lanes = lax.iota(jnp.int32, 16)
        p_lo = (wid * NPAIR) // NW
        p_hi = ((wid + 1) * NPAIR) // NW

        def one_pass(sids_h, pos_h, bnds_h, tab_h, tail_h, out_h):
            pltpu.sync_copy(sids_h, ids_v.at[pl.ds(0, B)])
            pltpu.sync_copy(pos_h, pos_v.at[pl.ds(0, B)])
            pltpu.sync_copy(bnds_h, bnd_v.at[pl.ds(0, NBPAD)])
            pltpu.sync_copy(tail_h, buft)

            def bounds(blk):
                v = bnd_v[pl.ds(blk, 16)]
                return v[0], v[1]

            def blk_dma(blk, buf, sem):
                blk_c = jnp.minimum(blk, NBLK - 1)
                off = pl.multiple_of(blk_c * BLKW, BLKW)
                pltpu.async_copy(tab_h.at[:, pl.ds(off, BLKW)],
                                 buf.at[pl.ds(0, D), :], sem)

            def blk_wait(buf, sem):
                pltpu.make_async_copy(tab_h.at[:, pl.ds(0, BLKW)],
                                      buf.at[pl.ds(0, D), :], sem).wait()

            def emit_row(posv, cnt):
                slot = jnp.bitwise_and(cnt, RING - 1)
                pltpu.async_copy(ring.at[pl.ds(slot, 1), :],
                                 out_h.at[pl.ds(posv, 1), :], semw)

                @pl.when(cnt >= RING - 1)
                def _():
                    pltpu.make_async_copy(out_h.at[pl.ds(0, 1), :],
                                          ring.at[pl.ds(0, 1), :],
                                          semw).wait()

            def consume(blk, buf, cnt0):
                lo, hi = bounds(blk)

                def bd(j, cnt):
                    idv = ids_v[pl.ds(j, 16)][0]
                    posv = pos_v[pl.ds(j, 16)][0]
                    col = jnp.bitwise_and(idv, BLKW - 1)
                    slot = jnp.bitwise_and(cnt, RING - 1)
                    for q in range(D // 16):
                        acc = jnp.zeros((16,), jnp.float32)
                        for l in range(16):
                            v = buf[16 * q + l, pl.ds(col, 16)]
                            acc = jnp.where(
                                lanes == l,
                                jnp.full((16,), v[0], jnp.float32), acc)
                        ring[slot, pl.ds(16 * q, 16)] = acc
                    emit_row(posv, cnt)
                    return cnt + 1

                return lax.fori_loop(lo, hi, bd, cnt0)

            blk_dma(2 * p_lo, bufa, sema)
            blk_dma(2 * p_lo + 1, bufb, semb)

            def pair(p, cnt):
                blk = 2 * p
                blk_wait(bufa, sema)
                cnt = consume(blk, bufa, cnt)
                blk_dma(blk + 2, bufa, sema)
                blk_wait(bufb, semb)
                cnt = consume(blk + 1, bufb, cnt)
                blk_dma(blk + 3, bufb, semb)
                return cnt

            cnt = lax.fori_loop(p_lo, p_hi, pair, 0)
            blk_wait(bufa, sema)
            blk_wait(bufb, semb)

            t_lo, t_hi = bounds(NBLK)
            t_hi = jnp.where(wid == NW - 1, t_hi, t_lo)

            def tbd(j, cnt):
                idv = ids_v[pl.ds(j, 16)][0]
                posv = pos_v[pl.ds(j, 16)][0]
                idrel = jnp.bitwise_and(idv, BLKW - 1)
                slot = jnp.bitwise_and(cnt, RING - 1)
                for q in range(D // 16):
                    ring[slot, pl.ds(16 * q, 16)] = buft[idrel,
                                                         pl.ds(16 * q, 16)]
                emit_row(posv, cnt)
                return cnt + 1

            cnt = lax.fori_loop(t_lo, t_hi, tbd, cnt)

            def drain(_, c):
                pltpu.make_async_copy(out_h.at[pl.ds(0, 1), :],
                                      ring.at[pl.ds(0, 1), :], semw).wait()
                return c

            lax.fori_loop(0, jnp.minimum(cnt, RING - 1), drain, 0)

        one_pass(su_h, pu_h, bu_h, ut_h, utail_h, uout_h)
        one_pass(si_h, pi_h, bi_h, it_h, itail_h, iout_h)

    return k(su, pu, bu, si, pi, bi, ut, it, utail, itail)


def _tc_towers(u_emb, i_emb, W_u, b_u, W_i, b_i):
    BLK = 2048

    def body(u_ref, i_ref, wu_ref, bu_ref, wi_ref, bi_ref, out_ref):
        u = jnp.dot(u_ref[...], wu_ref[...],
                    preferred_element_type=jnp.float32) + bu_ref[...]
        i = jnp.dot(i_ref[...], wi_ref[...],
                    preferred_element_type=jnp.float32) + bi_ref[...]
        u = jnp.maximum(u, 0.0)
        i = jnp.maximum(i, 0.0)
        out_ref[...] = jnp.sum(u * i, axis=1)

    return pl.pallas_call(
        body,
        grid=(B // BLK,),
        in_specs=[
            pl.BlockSpec((BLK, D), lambda g: (g, 0)),
            pl.BlockSpec((BLK, D), lambda g: (g, 0)),
            pl.BlockSpec((D, D), lambda g: (0, 0)),
            pl.BlockSpec((D,), lambda g: (0,)),
            pl.BlockSpec((D, D), lambda g: (0, 0)),
            pl.BlockSpec((D,), lambda g: (0,)),
        ],
        out_specs=pl.BlockSpec((BLK,), lambda g: (g,)),
        out_shape=jax.ShapeDtypeStruct((B,), jnp.float32),
    )(u_emb, i_emb, W_u, b_u, W_i, b_i)


def kernel(user_ids, item_ids, user_table, item_table, W_u, b_u, W_i, b_i):
    iota = lax.iota(jnp.int32, B)
    su, pu = lax.sort_key_val(user_ids, iota)
    si, pi = lax.sort_key_val(item_ids, iota)
    edges = lax.iota(jnp.int32, NBPAD) * BLKW
    bu = jnp.searchsorted(su, edges).astype(jnp.int32)
    bi = jnp.searchsorted(si, edges).astype(jnp.int32)
    utail = user_table[NBLK * BLKW:, :]
    itail = item_table[NBLK * BLKW:, :]
    u_emb, i_emb = _sc_scan_gather(su, pu, bu, si, pi, bi,
                                   user_table.T, item_table.T, utail, itail)
    return _tc_towers(u_emb, i_emb, W_u, b_u, W_i, b_i)


# speedup vs baseline: 1.6684x; 1.1844x over previous
import functools

import jax
import jax.numpy as jnp
from jax import lax
from jax.experimental import pallas as pl
from jax.experimental.pallas import tpu as pltpu
from jax.experimental.pallas import tpu_sc as plsc

B = 16384
D = 64
V = 1000000
NC = 2
NS = 16
NW = NC * NS
BLKW = 128
NBLK = V // BLKW
TAILW = V - NBLK * BLKW
NPAIR = NBLK // 2
NEDGE = NBLK + 2
NBPAD = 7840
RING = 128


def _sc_scan_gather(su, pu, bu, si, pi, bi, ut, it, utail, itail):
    @functools.partial(
        pl.kernel,
        mesh=plsc.VectorSubcoreMesh(core_axis_name="c", subcore_axis_name="s"),
        out_type=[
            jax.ShapeDtypeStruct((B, D), jnp.float32),
            jax.ShapeDtypeStruct((B, D), jnp.float32),
        ],
        scratch_types=[
            pltpu.VMEM((B + 16,), jnp.int32),
            pltpu.VMEM((B + 16,), jnp.int32),
            pltpu.VMEM((NBPAD + 16,), jnp.int32),
            pltpu.VMEM((D + 1, BLKW), jnp.float32),
            pltpu.VMEM((D + 1, BLKW), jnp.float32),
            pltpu.VMEM((TAILW, D), jnp.float32),
            pltpu.VMEM((RING, D), jnp.float32),
            pltpu.SemaphoreType.DMA,
            pltpu.SemaphoreType.DMA,
            pltpu.SemaphoreType.DMA,
        ],
    )
    def k(su_h, pu_h, bu_h, si_h, pi_h, bi_h, ut_h, it_h, utail_h, itail_h,
          uout_h, iout_h,
          ids_v, pos_v, bnd_v, bufa, bufb, buft, ring, sema, semb, semw):
        wid = lax.axis_index("s") * NC + lax.axis_index("c")
        lanes = lax.iota(jnp.int32, 16)
        p_lo = (wid * NPAIR) // NW
        p_hi = ((wid + 1) * NPAIR) // NW

        def one_pass(sids_h, pos_h, bnds_h, tab_h, tail_h, out_h):
            pltpu.sync_copy(sids_h, ids_v.at[pl.ds(0, B)])
            pltpu.sync_copy(pos_h, pos_v.at[pl.ds(0, B)])
            pltpu.sync_copy(bnds_h, bnd_v.at[pl.ds(0, NBPAD)])
            pltpu.sync_copy(tail_h, buft)

            def bounds(blk):
                v = bnd_v[pl.ds(blk, 16)]
                return v[0], v[1]

            def blk_dma(blk, buf, sem):
                blk_c = jnp.minimum(blk, NBLK - 1)
                off = pl.multiple_of(blk_c * BLKW, BLKW)
                pltpu.async_copy(tab_h.at[:, pl.ds(off, BLKW)],
                                 buf.at[pl.ds(0, D), :], sem)

            def blk_wait(buf, sem):
                pltpu.make_async_copy(tab_h.at[:, pl.ds(0, BLKW)],
                                      buf.at[pl.ds(0, D), :], sem).wait()

            def emit_row(posv, cnt):
                slot = jnp.bitwise_and(cnt, RING - 1)
                pltpu.async_copy(ring.at[pl.ds(slot, 1), :],
                                 out_h.at[pl.ds(posv, 1), :], semw)

                @pl.when(cnt >= RING - 1)
                def _():
                    pltpu.make_async_copy(out_h.at[pl.ds(0, 1), :],
                                          ring.at[pl.ds(0, 1), :],
                                          semw).wait()

            def consume(blk, buf, cnt0):
                lo, hi = bounds(blk)

                def bd(j, cnt):
                    idv = ids_v[pl.ds(j, 16)][0]
                    posv = pos_v[pl.ds(j, 16)][0]
                    col = jnp.bitwise_and(idv, BLKW - 1)
                    slot = jnp.bitwise_and(cnt, RING - 1)
                    for q in range(D // 16):
                        acc = jnp.zeros((16,), jnp.float32)
                        for l in range(16):
                            v = buf[16 * q + l, pl.ds(col, 16)]
                            acc = jnp.where(
                                lanes == l,
                                jnp.full((16,), v[0], jnp.float32), acc)
                        ring[slot, pl.ds(16 * q, 16)] = acc
                    emit_row(posv, cnt)
                    return cnt + 1

                return lax.fori_loop(lo, hi, bd, cnt0)

            blk_dma(2 * p_lo, bufa, sema)
            blk_dma(2 * p_lo + 1, bufb, semb)

            def pair(p, cnt):
                blk = 2 * p
                blk_wait(bufa, sema)
                cnt = consume(blk, bufa, cnt)
                blk_dma(blk + 2, bufa, sema)
                blk_wait(bufb, semb)
                cnt = consume(blk + 1, bufb, cnt)
                blk_dma(blk + 3, bufb, semb)
                return cnt

            cnt = lax.fori_loop(p_lo, p_hi, pair, 0)
            blk_wait(bufa, sema)
            blk_wait(bufb, semb)

            t_lo, t_hi = bounds(NBLK)
            t_hi = jnp.where(wid == NW - 1, t_hi, t_lo)

            def tbd(j, cnt):
                idv = ids_v[pl.ds(j, 16)][0]
                posv = pos_v[pl.ds(j, 16)][0]
                idrel = jnp.bitwise_and(idv, BLKW - 1)
                slot = jnp.bitwise_and(cnt, RING - 1)
                for q in range(D // 16):
                    ring[slot, pl.ds(16 * q, 16)] = buft[idrel,
                                                         pl.ds(16 * q, 16)]
                emit_row(posv, cnt)
                return cnt + 1

            cnt = lax.fori_loop(t_lo, t_hi, tbd, cnt)

            def drain(_, c):
                pltpu.make_async_copy(out_h.at[pl.ds(0, 1), :],
                                      ring.at[pl.ds(0, 1), :], semw).wait()
                return c

            lax.fori_loop(0, jnp.minimum(cnt, RING - 1), drain, 0)

        one_pass(su_h, pu_h, bu_h, ut_h, utail_h, uout_h)
        one_pass(si_h, pi_h, bi_h, it_h, itail_h, iout_h)

    return k(su, pu, bu, si, pi, bi, ut, it, utail, itail)


def _tc_towers(u_emb, i_emb, W_u, b_u, W_i, b_i):
    BLK = 2048

    def body(u_ref, i_ref, wu_ref, bu_ref, wi_ref, bi_ref, out_ref):
        u = jnp.dot(u_ref[...], wu_ref[...],
                    preferred_element_type=jnp.float32) + bu_ref[...]
        i = jnp.dot(i_ref[...], wi_ref[...],
                    preferred_element_type=jnp.float32) + bi_ref[...]
        u = jnp.maximum(u, 0.0)
        i = jnp.maximum(i, 0.0)
        out_ref[...] = jnp.sum(u * i, axis=1)

    return pl.pallas_call(
        body,
        grid=(B // BLK,),
        in_specs=[
            pl.BlockSpec((BLK, D), lambda g: (g, 0)),
            pl.BlockSpec((BLK, D), lambda g: (g, 0)),
            pl.BlockSpec((D, D), lambda g: (0, 0)),
            pl.BlockSpec((D,), lambda g: (0,)),
            pl.BlockSpec((D, D), lambda g: (0, 0)),
            pl.BlockSpec((D,), lambda g: (0,)),
        ],
        out_specs=pl.BlockSpec((BLK,), lambda g: (g,)),
        out_shape=jax.ShapeDtypeStruct((B,), jnp.float32),
    )(u_emb, i_emb, W_u, b_u, W_i, b_i)


def kernel(user_ids, item_ids, user_table, item_table, W_u, b_u, W_i, b_i):
    iota = lax.iota(jnp.int32, B)
    su, pu = lax.sort_key_val(user_ids, iota)
    si, pi = lax.sort_key_val(item_ids, iota)
    edges = lax.iota(jnp.int32, NBPAD) * BLKW
    bu = jnp.searchsorted(su, edges, method="sort").astype(jnp.int32)
    bi = jnp.searchsorted(si, edges, method="sort").astype(jnp.int32)
    utail = user_table[NBLK * BLKW:, :]
    itail = item_table[NBLK * BLKW:, :]
    u_emb, i_emb = _sc_scan_gather(su, pu, bu, si, pi, bi,
                                   user_table.T, item_table.T, utail, itail)
    return _tc_towers(u_emb, i_emb, W_u, b_u, W_i, b_i)


# speedup vs baseline: 1.9883x; 1.1918x over previous
import functools

import jax
import jax.numpy as jnp
import numpy as np
from jax import lax
from jax.experimental import pallas as pl
from jax.experimental.pallas import tpu as pltpu
from jax.experimental.pallas import tpu_sc as plsc

B = 16384
D = 64
V = 1000000
NC = 2
NS = 16
NW = NC * NS
BLKW = 128
NBLK = V // BLKW
TAILW = V - NBLK * BLKW
RING = 128

_EDGES = np.asarray(
    [((w * NBLK) // NW) * BLKW for w in range(NW)] + [V], dtype=np.int32)


def _sc_scan_gather(su, pu, bu, si, pi, bi, ut, it, utail, itail):
    @functools.partial(
        pl.kernel,
        mesh=plsc.VectorSubcoreMesh(core_axis_name="c", subcore_axis_name="s"),
        out_type=[
            jax.ShapeDtypeStruct((B, D), jnp.float32),
            jax.ShapeDtypeStruct((B, D), jnp.float32),
        ],
        scratch_types=[
            pltpu.VMEM((B + 16,), jnp.int32),
            pltpu.VMEM((B + 16,), jnp.int32),
            pltpu.VMEM((48,), jnp.int32),
            pltpu.VMEM((D + 1, BLKW), jnp.float32),
            pltpu.VMEM((TAILW, D), jnp.float32),
            pltpu.VMEM((RING, D), jnp.float32),
            pltpu.SemaphoreType.DMA,
            pltpu.SemaphoreType.DMA,
        ],
    )
    def k(su_h, pu_h, bu_h, si_h, pi_h, bi_h, ut_h, it_h, utail_h, itail_h,
          uout_h, iout_h,
          ids_v, pos_v, bnd_v, bufa, buft, ring, sema, semw):
        wid = lax.axis_index("s") * NC + lax.axis_index("c")
        lanes = lax.iota(jnp.int32, 16)

        def one_pass(sids_h, pos_h, bnds_h, tab_h, tail_h, out_h):
            pltpu.sync_copy(sids_h, ids_v.at[pl.ds(0, B)])
            pltpu.sync_copy(pos_h, pos_v.at[pl.ds(0, B)])
            pltpu.sync_copy(bnds_h, bnd_v)
            pltpu.sync_copy(tail_h, buft)
            segv = bnd_v[pl.ds(wid, 16)]
            seg_lo, seg_hi = segv[0], segv[1]

            def emit_row(posv, cnt):
                slot = jnp.bitwise_and(cnt, RING - 1)
                pltpu.async_copy(ring.at[pl.ds(slot, 1), :],
                                 out_h.at[pl.ds(posv, 1), :], semw)

                @pl.when(cnt >= RING - 1)
                def _():
                    pltpu.make_async_copy(out_h.at[pl.ds(0, 1), :],
                                          ring.at[pl.ds(0, 1), :],
                                          semw).wait()

            def bd(j, carry):
                cur, cnt = carry
                idv = ids_v[pl.ds(j, 16)][0]
                posv = pos_v[pl.ds(j, 16)][0]
                blk = lax.shift_right_logical(idv, 7)
                col = jnp.bitwise_and(idv, BLKW - 1)
                slot = jnp.bitwise_and(cnt, RING - 1)

                @pl.when(jnp.logical_and(blk != cur, blk < NBLK))
                def _():
                    off = pl.multiple_of(blk * BLKW, BLKW)
                    pltpu.sync_copy(tab_h.at[:, pl.ds(off, BLKW)],
                                    bufa.at[pl.ds(0, D), :])

                @pl.when(blk < NBLK)
                def _():
                    for q in range(D // 16):
                        acc = jnp.zeros((16,), jnp.float32)
                        for l in range(16):
                            v = bufa[16 * q + l, pl.ds(col, 16)]
                            acc = jnp.where(
                                lanes == l,
                                jnp.full((16,), v[0], jnp.float32), acc)
                        ring[slot, pl.ds(16 * q, 16)] = acc

                @pl.when(blk >= NBLK)
                def _():
                    for q in range(D // 16):
                        ring[slot, pl.ds(16 * q, 16)] = buft[col,
                                                             pl.ds(16 * q,
                                                                   16)]

                emit_row(posv, cnt)
                return blk, cnt + 1

            _, cnt = lax.fori_loop(seg_lo, seg_hi, bd, (-1, 0))

            def drain(_, c):
                pltpu.make_async_copy(out_h.at[pl.ds(0, 1), :],
                                      ring.at[pl.ds(0, 1), :], semw).wait()
                return c

            lax.fori_loop(0, jnp.minimum(cnt, RING - 1), drain, 0)

        one_pass(su_h, pu_h, bu_h, ut_h, utail_h, uout_h)
        one_pass(si_h, pi_h, bi_h, it_h, itail_h, iout_h)

    return k(su, pu, bu, si, pi, bi, ut, it, utail, itail)


def _tc_towers(u_emb, i_emb, W_u, b_u, W_i, b_i):
    BLK = 2048

    def body(u_ref, i_ref, wu_ref, bu_ref, wi_ref, bi_ref, out_ref):
        u = jnp.dot(u_ref[...], wu_ref[...],
                    preferred_element_type=jnp.float32) + bu_ref[...]
        i = jnp.dot(i_ref[...], wi_ref[...],
                    preferred_element_type=jnp.float32) + bi_ref[...]
        u = jnp.maximum(u, 0.0)
        i = jnp.maximum(i, 0.0)
        out_ref[...] = jnp.sum(u * i, axis=1)

    return pl.pallas_call(
        body,
        grid=(B // BLK,),
        in_specs=[
            pl.BlockSpec((BLK, D), lambda g: (g, 0)),
            pl.BlockSpec((BLK, D), lambda g: (g, 0)),
            pl.BlockSpec((D, D), lambda g: (0, 0)),
            pl.BlockSpec((D,), lambda g: (0,)),
            pl.BlockSpec((D, D), lambda g: (0, 0)),
            pl.BlockSpec((D,), lambda g: (0,)),
        ],
        out_specs=pl.BlockSpec((BLK,), lambda g: (g,)),
        out_shape=jax.ShapeDtypeStruct((B,), jnp.float32),
    )(u_emb, i_emb, W_u, b_u, W_i, b_i)


def kernel(user_ids, item_ids, user_table, item_table, W_u, b_u, W_i, b_i):
    iota = lax.iota(jnp.int32, B)
    su, pu = lax.sort_key_val(user_ids, iota)
    si, pi = lax.sort_key_val(item_ids, iota)
    edges = jnp.asarray(_EDGES)
    bu = jnp.pad(jnp.sum(su[None, :] < edges[:, None],
                         axis=1).astype(jnp.int32), (0, 15))
    bi = jnp.pad(jnp.sum(si[None, :] < edges[:, None],
                         axis=1).astype(jnp.int32), (0, 15))
    utail = user_table[NBLK * BLKW:, :]
    itail = item_table[NBLK * BLKW:, :]
    u_emb, i_emb = _sc_scan_gather(su, pu, bu, si, pi, bi,
                                   user_table.T, item_table.T, utail, itail)
    return _tc_towers(u_emb, i_emb, W_u, b_u, W_i, b_i)


# speedup vs baseline: 2.0163x; 1.0141x over previous
import functools

import jax
import jax.numpy as jnp
import numpy as np
from jax import lax
from jax.experimental import pallas as pl
from jax.experimental.pallas import tpu as pltpu
from jax.experimental.pallas import tpu_sc as plsc

B = 16384
D = 64
V = 1000000
NC = 2
NS = 16
NW = NC * NS
BLKW = 128
NBLK = V // BLKW
TAILW = V - NBLK * BLKW
RING = 128

_EDGES = np.asarray(
    [((w * NBLK) // NW) * BLKW for w in range(NW)] + [V], dtype=np.int32)


def _sc_scan_gather(su, pu, bu, si, pi, bi, ut, it, utail, itail):
    @functools.partial(
        pl.kernel,
        mesh=plsc.VectorSubcoreMesh(core_axis_name="c", subcore_axis_name="s"),
        out_type=[
            jax.ShapeDtypeStruct((B, D), jnp.float32),
            jax.ShapeDtypeStruct((B, D), jnp.float32),
        ],
        scratch_types=[
            pltpu.VMEM((B + 16,), jnp.int32),
            pltpu.VMEM((B + 16,), jnp.int32),
            pltpu.VMEM((48,), jnp.int32),
            pltpu.VMEM((D + 1, BLKW), jnp.float32),
            pltpu.VMEM((D + 1, BLKW), jnp.float32),
            pltpu.VMEM((TAILW, D), jnp.float32),
            pltpu.VMEM((RING, D), jnp.float32),
            pltpu.SemaphoreType.DMA,
            pltpu.SemaphoreType.DMA,
        ],
    )
    def k(su_h, pu_h, bu_h, si_h, pi_h, bi_h, ut_h, it_h, utail_h, itail_h,
          uout_h, iout_h,
          ids_v, pos_v, bnd_v, bufa, bufb, buft, ring, semb, semw):
        wid = lax.axis_index("s") * NC + lax.axis_index("c")
        lanes = lax.iota(jnp.int32, 16)

        def one_pass(sids_h, pos_h, bnds_h, tab_h, tail_h, out_h):
            pltpu.sync_copy(sids_h, ids_v.at[pl.ds(0, B)])
            pltpu.sync_copy(pos_h, pos_v.at[pl.ds(0, B)])
            pltpu.sync_copy(bnds_h, bnd_v)
            pltpu.sync_copy(tail_h, buft)
            segv = bnd_v[pl.ds(wid, 16)]
            seg_lo, seg_hi = segv[0], segv[1]

            def emit_row(posv, cnt):
                slot = jnp.bitwise_and(cnt, RING - 1)
                pltpu.async_copy(ring.at[pl.ds(slot, 1), :],
                                 out_h.at[pl.ds(posv, 1), :], semw)

                @pl.when(cnt >= RING - 1)
                def _():
                    pltpu.make_async_copy(out_h.at[pl.ds(0, 1), :],
                                          ring.at[pl.ds(0, 1), :],
                                          semw).wait()

            def pf_wait():
                pltpu.make_async_copy(tab_h.at[:, pl.ds(0, BLKW)],
                                      bufb.at[pl.ds(0, D), :], semb).wait()

            def extract(buf, col, slot):
                for q in range(D // 16):
                    acc = jnp.zeros((16,), jnp.float32)
                    for l in range(16):
                        v = buf[16 * q + l, pl.ds(col, 16)]
                        acc = jnp.where(
                            lanes == l,
                            jnp.full((16,), v[0], jnp.float32), acc)
                    ring[slot, pl.ds(16 * q, 16)] = acc

            def bd(j, carry):
                blka, blkb, pend, cnt = carry
                idv = ids_v[pl.ds(j, 16)][0]
                posv = pos_v[pl.ds(j, 16)][0]
                blk = lax.shift_right_logical(idv, 7)
                col = jnp.bitwise_and(idv, BLKW - 1)
                slot = jnp.bitwise_and(cnt, RING - 1)
                full = blk < NBLK
                hit_a = jnp.logical_and(blk == blka, full)
                hit_b = jnp.logical_and(blk == blkb, full)
                miss = jnp.logical_and(
                    full, jnp.logical_not(jnp.logical_or(hit_a, hit_b)))

                @pl.when(jnp.logical_and(miss, pend == 1))
                def _():
                    pf_wait()

                @pl.when(miss)
                def _():
                    off = pl.multiple_of(blk * BLKW, BLKW)
                    pltpu.sync_copy(tab_h.at[:, pl.ds(off, BLKW)],
                                    bufa.at[pl.ds(0, D), :])
                    nxt = jnp.minimum(blk + 1, NBLK - 1)
                    offn = pl.multiple_of(nxt * BLKW, BLKW)
                    pltpu.async_copy(tab_h.at[:, pl.ds(offn, BLKW)],
                                     bufb.at[pl.ds(0, D), :], semb)

                @pl.when(jnp.logical_and(hit_b, pend == 1))
                def _():
                    pf_wait()

                @pl.when(jnp.logical_or(hit_a, miss))
                def _():
                    extract(bufa, col, slot)

                @pl.when(hit_b)
                def _():
                    extract(bufb, col, slot)

                @pl.when(jnp.logical_not(full))
                def _():
                    for q in range(D // 16):
                        ring[slot, pl.ds(16 * q, 16)] = buft[col,
                                                             pl.ds(16 * q,
                                                                   16)]

                emit_row(posv, cnt)
                blka2 = jnp.where(miss, blk, blka)
                blkb2 = jnp.where(miss, blk + 1, blkb)
                pend2 = jnp.where(miss, 1, jnp.where(hit_b, 0, pend))
                return blka2, blkb2, pend2, cnt + 1

            _, _, pend, cnt = lax.fori_loop(seg_lo, seg_hi, bd,
                                            (-1, -1, 0, 0))

            @pl.when(pend == 1)
            def _():
                pf_wait()

            def drain(_, c):
                pltpu.make_async_copy(out_h.at[pl.ds(0, 1), :],
                                      ring.at[pl.ds(0, 1), :], semw).wait()
                return c

            lax.fori_loop(0, jnp.minimum(cnt, RING - 1), drain, 0)

        one_pass(su_h, pu_h, bu_h, ut_h, utail_h, uout_h)
        one_pass(si_h, pi_h, bi_h, it_h, itail_h, iout_h)

    return k(su, pu, bu, si, pi, bi, ut, it, utail, itail)


def _tc_towers(u_emb, i_emb, W_u, b_u, W_i, b_i):
    BLK = 2048

    def body(u_ref, i_ref, wu_ref, bu_ref, wi_ref, bi_ref, out_ref):
        u = jnp.dot(u_ref[...], wu_ref[...],
                    preferred_element_type=jnp.float32) + bu_ref[...]
        i = jnp.dot(i_ref[...], wi_ref[...],
                    preferred_element_type=jnp.float32) + bi_ref[...]
        u = jnp.maximum(u, 0.0)
        i = jnp.maximum(i, 0.0)
        out_ref[...] = jnp.sum(u * i, axis=1)

    return pl.pallas_call(
        body,
        grid=(B // BLK,),
        in_specs=[
            pl.BlockSpec((BLK, D), lambda g: (g, 0)),
            pl.BlockSpec((BLK, D), lambda g: (g, 0)),
            pl.BlockSpec((D, D), lambda g: (0, 0)),
            pl.BlockSpec((D,), lambda g: (0,)),
            pl.BlockSpec((D, D), lambda g: (0, 0)),
            pl.BlockSpec((D,), lambda g: (0,)),
        ],
        out_specs=pl.BlockSpec((BLK,), lambda g: (g,)),
        out_shape=jax.ShapeDtypeStruct((B,), jnp.float32),
    )(u_emb, i_emb, W_u, b_u, W_i, b_i)


def kernel(user_ids, item_ids, user_table, item_table, W_u, b_u, W_i, b_i):
    iota = lax.iota(jnp.int32, B)
    su, pu = lax.sort_key_val(user_ids, iota)
    si, pi = lax.sort_key_val(item_ids, iota)
    edges = jnp.asarray(_EDGES)
    bu = jnp.pad(jnp.sum(su[None, :] < edges[:, None],
                         axis=1).astype(jnp.int32), (0, 15))
    bi = jnp.pad(jnp.sum(si[None, :] < edges[:, None],
                         axis=1).astype(jnp.int32), (0, 15))
    utail = user_table[NBLK * BLKW:, :]
    itail = item_table[NBLK * BLKW:, :]
    u_emb, i_emb = _sc_scan_gather(su, pu, bu, si, pi, bi,
                                   user_table.T, item_table.T, utail, itail)
    return _tc_towers(u_emb, i_emb, W_u, b_u, W_i, b_i)
